# Initial kernel scaffold; baseline (speedup 1.0000x reference)
import jax
import jax.numpy as jnp
from jax.experimental import pallas as pl


def kernel(embeddings, eot_idx, edge_index, W1, b1, W2, b2):
    raise NotImplementedError("write your pallas kernel here")


# speedup vs baseline: 6.1356x; 6.1356x over previous
import functools

import jax
import jax.numpy as jnp
from jax import lax
from jax.experimental import pallas as pl
from jax.experimental.pallas import tpu as pltpu
from jax.experimental.pallas import tpu_sc as plsc

NC, NS = 2, 16
NW = NC * NS
CH = 128
GCH = 64
RB = 2048
NB = 2000


def _mesh():
    return plsc.VectorSubcoreMesh(core_axis_name="c", subcore_axis_name="s")


def _sc_gather_and_degree(embflat, gidx, dst_p, zfull, ones_rows, NPAD, D):
    EP = dst_p.shape[0]
    EPW = EP // NW
    NCHUNK = EPW // CH
    RPW = NPAD // NW
    TPT = NPAD // NS

    @functools.partial(
        pl.kernel,
        mesh=_mesh(),
        out_type=(
            jax.ShapeDtypeStruct((NPAD, D), jnp.float32),
            jax.ShapeDtypeStruct((NC, NPAD, D), jnp.float32),
        ),
        scratch_types=[
            pltpu.VMEM((GCH,), jnp.int32),
            pltpu.VMEM((GCH, D), jnp.float32),
            pltpu.VMEM((CH,), jnp.int32),
            pltpu.VMEM((CH, D), jnp.float32),
            pltpu.VMEM_SHARED((NPAD, D), jnp.float32),
        ],
    )
    def k(emb_h, gidx_h, dst_h, z_h, ones_h, eot_h, degp_h,
          idx_v, rows_v, didx_v, ones_v, deg_sh):
        c = lax.axis_index("c")
        s = lax.axis_index("s")
        w = c * NS + s
        pltpu.sync_copy(z_h.at[pl.ds(s * TPT, TPT)],
                        deg_sh.at[pl.ds(s * TPT, TPT)])
        pltpu.sync_copy(ones_h, ones_v)
        plsc.subcore_barrier()

        @pl.loop(0, RPW // GCH)
        def _(j):
            off = w * RPW + j * GCH
            pltpu.sync_copy(gidx_h.at[pl.ds(off, GCH)], idx_v)
            pltpu.sync_copy(emb_h.at[idx_v], rows_v)
            pltpu.sync_copy(rows_v, eot_h.at[pl.ds(off, GCH)])

        @pl.loop(0, NCHUNK)
        def _(j):
            off = w * EPW + j * CH
            pltpu.sync_copy(dst_h.at[pl.ds(off, CH)], didx_v)
            pltpu.sync_copy(ones_v, deg_sh.at[didx_v], add=True)

        plsc.subcore_barrier()
        pltpu.sync_copy(deg_sh.at[pl.ds(s * TPT, TPT)],
                        degp_h.at[c].at[pl.ds(s * TPT, TPT)])

    return k(embflat, gidx, dst_p, zfull, ones_rows)


def _sc_edge_spmm(hs, src_p, dst_p, z128, NPAD, D):
    EP = dst_p.shape[0]
    EPW = EP // NW
    NCHUNK = EPW // CH
    TPT = NPAD // NS

    @functools.partial(
        pl.kernel,
        mesh=_mesh(),
        out_type=jax.ShapeDtypeStruct((NC, NPAD, D), jnp.float32),
        scratch_types=[
            pltpu.VMEM((CH,), jnp.int32),
            pltpu.VMEM((CH,), jnp.int32),
            pltpu.VMEM((CH, D), jnp.float32),
            pltpu.VMEM_SHARED((NPAD, D), jnp.float32),
        ],
    )
    def k(hs_h, src_h, dst_h, z_h, aggp_h, sidx_v, didx_v, rows_v, agg_sh):
        c = lax.axis_index("c")
        s = lax.axis_index("s")
        w = c * NS + s
        pltpu.sync_copy(z_h.at[pl.ds(s * TPT, TPT)],
                        agg_sh.at[pl.ds(s * TPT, TPT)])
        plsc.subcore_barrier()

        @pl.loop(0, NCHUNK)
        def _(j):
            off = w * EPW + j * CH
            pltpu.sync_copy(src_h.at[pl.ds(off, CH)], sidx_v)
            pltpu.sync_copy(dst_h.at[pl.ds(off, CH)], didx_v)
            pltpu.sync_copy(hs_h.at[sidx_v], rows_v)
            pltpu.sync_copy(rows_v, agg_sh.at[didx_v], add=True)

        plsc.subcore_barrier()
        pltpu.sync_copy(agg_sh.at[pl.ds(s * TPT, TPT)],
                        aggp_h.at[c].at[pl.ds(s * TPT, TPT)])

    return k(hs, src_p, dst_p, z128)


def _tc_prescale(degp, eot):
    NPAD, D = eot.shape
    NI = NPAD // RB

    def body(degp_ref, eot_ref, hs_ref):
        deg = degp_ref[0, :, 0:1] + degp_ref[1, :, 0:1] + 1.0
        hs_ref[...] = eot_ref[...] * lax.rsqrt(deg)

    return pl.pallas_call(
        body,
        grid=(NI,),
        in_specs=[
            pl.BlockSpec((NC, RB, D), lambda i: (0, i, 0)),
            pl.BlockSpec((RB, D), lambda i: (i, 0)),
        ],
        out_specs=pl.BlockSpec((RB, D), lambda i: (i, 0)),
        out_shape=jax.ShapeDtypeStruct((NPAD, D), jnp.float32),
    )(degp, eot)


def _tc_layer(degp, h, aggp, W, b, relu, rescale):
    NPAD, D = h.shape
    NI = NPAD // RB
    b2d = b.reshape(1, D)
    n_out = 2 if rescale else 1

    def body(degp_ref, h_ref, aggp_ref, w_ref, b_ref, *outs):
        deg = degp_ref[0, :, 0:1] + degp_ref[1, :, 0:1] + 1.0
        rs = lax.rsqrt(deg)
        pre = (aggp_ref[0] + aggp_ref[1]) * rs + h_ref[...] * (1.0 / deg)
        out = jnp.dot(pre, w_ref[...],
                      preferred_element_type=jnp.float32) + b_ref[...]
        if relu:
            out = jnp.maximum(out, 0.0)
        outs[0][...] = out
        if rescale:
            outs[1][...] = out * rs

    res = pl.pallas_call(
        body,
        grid=(NI,),
        in_specs=[
            pl.BlockSpec((NC, RB, D), lambda i: (0, i, 0)),
            pl.BlockSpec((RB, D), lambda i: (i, 0)),
            pl.BlockSpec((NC, RB, D), lambda i: (0, i, 0)),
            pl.BlockSpec((D, D), lambda i: (0, 0)),
            pl.BlockSpec((1, D), lambda i: (0, 0)),
        ],
        out_specs=tuple(pl.BlockSpec((RB, D), lambda i: (i, 0))
                        for _ in range(n_out)),
        out_shape=tuple(jax.ShapeDtypeStruct((NPAD, D), jnp.float32)
                        for _ in range(n_out)),
    )(degp, h, aggp, W, b2d)
    return res if rescale else (res[0], None)


def _tc_assemble(embeddings, eidx_col, gcn):
    S, N, D = embeddings.shape
    NJ = N // NB
    embflat = embeddings.reshape(S * N, D)

    def body(eidx_ref, gcn_ref, emb_ref, out_ref):
        s = pl.program_id(1)
        m = (eidx_ref[...] == s).astype(jnp.float32)
        out_ref[...] = emb_ref[...] + gcn_ref[...] * m

    out = pl.pallas_call(
        body,
        grid=(NJ, S),
        in_specs=[
            pl.BlockSpec((NB, 1), lambda j, s: (j, 0)),
            pl.BlockSpec((NB, D), lambda j, s: (j, 0)),
            pl.BlockSpec((NB, D), lambda j, s: (s * NJ + j, 0)),
        ],
        out_specs=pl.BlockSpec((NB, D), lambda j, s: (s * NJ + j, 0)),
        out_shape=jax.ShapeDtypeStruct((S * N, D), jnp.float32),
    )(eidx_col, gcn, embflat)
    return out.reshape(S, N, D)


def kernel(embeddings, eot_idx, edge_index, W1, b1, W2, b2):
    S, N, D = embeddings.shape
    E = edge_index.shape[1]

    NPAD = ((N + 8 * NW - 1) // (8 * NW)) * (8 * NW)
    while NPAD % NS or (NPAD // NW) % GCH or NPAD % RB:
        NPAD += 8 * NW
    EP = ((E + NW * CH - 1) // (NW * CH)) * (NW * CH)

    eot_idx = eot_idx.astype(jnp.int32)
    ar = jnp.arange(N, dtype=jnp.int32)
    gidx = jnp.concatenate(
        [eot_idx * N + ar, jnp.zeros((NPAD - N,), jnp.int32)])
    pad_e = jnp.full((EP - E,), N, jnp.int32)
    src_p = jnp.concatenate([edge_index[0].astype(jnp.int32), pad_e])
    dst_p = jnp.concatenate([edge_index[1].astype(jnp.int32), pad_e])

    z128 = jnp.zeros((NPAD, D), jnp.float32)
    ones_rows = jnp.ones((CH, D), jnp.float32)

    embflat = embeddings.reshape(S * N, D)
    eot, degp = _sc_gather_and_degree(embflat, gidx, dst_p, z128, ones_rows,
                                      NPAD, D)

    hs0 = _tc_prescale(degp, eot)
    aggp1 = _sc_edge_spmm(hs0, src_p, dst_p, z128, NPAD, D)
    h1, h1s = _tc_layer(degp, eot, aggp1, W1, b1, relu=True, rescale=True)
    aggp2 = _sc_edge_spmm(h1s, src_p, dst_p, z128, NPAD, D)
    out2, _ = _tc_layer(degp, h1, aggp2, W2, b2, relu=False, rescale=False)

    return _tc_assemble(embeddings, eot_idx.reshape(N, 1), out2[:N])
